# baseline (speedup 1.0000x reference)
import functools

import jax
import jax.numpy as jnp
from jax import lax
from jax.experimental import pallas as pl
from jax.experimental.pallas import tpu as pltpu
from jax.experimental.pallas import tpu_sc as plsc

N = 10000
E = 320000
D = 128
NOUT = 64
NGRAPH = 128

NC = 2
NS = 16
NW = NC * NS

NP = 10240
RPT = NP // NS
EPW = E // NW
CH = 80
NITER = EPW // CH
G = 6

_HIGH = jax.lax.Precision.DEFAULT


def _dot(a, b):
    return jax.lax.dot(a, b, precision=_HIGH, preferred_element_type=jnp.float32)


def _layer1_body(x_ref, w11, b11, w12, b12, w13, b13, wc, lin_ref, y_ref):
    xb = x_ref[...]
    a = _dot(xb, w11[...]) + b11[...]
    m2 = _dot(xb, w12[...]) + b12[...]
    m3 = _dot(xb, w13[...]) + b13[...]
    lin_ref[...] = a + m2 * m3
    y_ref[...] = _dot(xb, wc[...])


def _mid_body(lin_ref, agg_ref, w1, b1, w2, b2, w3, b3, wc, lin_o, y_o):
    h = jnp.tanh(lin_ref[...] + agg_ref[0] + agg_ref[1])
    a = _dot(h, w1[...]) + b1[...]
    m2 = _dot(h, w2[...]) + b2[...]
    m3 = _dot(h, w3[...]) + b3[...]
    lin_o[...] = a + m2 * m3
    y_o[...] = _dot(h, wc[...])


def _pool_body(lin_ref, agg_ref, batch_ref, wf, bf, out_ref, pooled):
    i = pl.program_id(0)
    h = jnp.tanh(lin_ref[...] + agg_ref[0] + agg_ref[1])
    b = batch_ref[...]
    gids = jax.lax.broadcasted_iota(jnp.int32, (1, NGRAPH), 1)
    onehot = (b == gids).astype(jnp.float32)
    part = jax.lax.dot_general(
        onehot, h, (((0,), (0,)), ((), ())),
        precision=_HIGH, preferred_element_type=jnp.float32)

    @pl.when(i == 0)
    def _():
        pooled[...] = part

    @pl.when(i > 0)
    def _():
        pooled[...] = pooled[...] + part

    @pl.when(i == pl.num_programs(0) - 1)
    def _():
        out_ref[...] = _dot(pooled[...], wf[...]) + bf[...]


_BN = 2048


def _full(shape):
    return pl.BlockSpec(shape, lambda i: (0,) * len(shape))


def _layer1(xp, w11, b11, w12, b12, w13, b13, wc):
    grid = NP // _BN
    return pl.pallas_call(
        _layer1_body,
        grid=(grid,),
        in_specs=[
            pl.BlockSpec((_BN, D), lambda i: (i, 0)),
            _full((D, NOUT)), _full((1, NOUT)),
            _full((D, NOUT)), _full((1, NOUT)),
            _full((D, NOUT)), _full((1, NOUT)),
            _full((D, NOUT)),
        ],
        out_specs=[
            pl.BlockSpec((_BN, NOUT), lambda i: (i, 0)),
            pl.BlockSpec((_BN, NOUT), lambda i: (i, 0)),
        ],
        out_shape=[
            jax.ShapeDtypeStruct((NP, NOUT), jnp.float32),
            jax.ShapeDtypeStruct((NP, NOUT), jnp.float32),
        ],
    )(xp, w11, b11, w12, b12, w13, b13, wc)


def _mid(lin, agg, w1, b1, w2, b2, w3, b3, wc):
    grid = NP // _BN
    return pl.pallas_call(
        _mid_body,
        grid=(grid,),
        in_specs=[
            pl.BlockSpec((_BN, NOUT), lambda i: (i, 0)),
            pl.BlockSpec((NC, _BN, NOUT), lambda i: (0, i, 0)),
            _full((NOUT, NOUT)), _full((1, NOUT)),
            _full((NOUT, NOUT)), _full((1, NOUT)),
            _full((NOUT, NOUT)), _full((1, NOUT)),
            _full((NOUT, NOUT)),
        ],
        out_specs=[
            pl.BlockSpec((_BN, NOUT), lambda i: (i, 0)),
            pl.BlockSpec((_BN, NOUT), lambda i: (i, 0)),
        ],
        out_shape=[
            jax.ShapeDtypeStruct((NP, NOUT), jnp.float32),
            jax.ShapeDtypeStruct((NP, NOUT), jnp.float32),
        ],
    )(lin, agg, w1, b1, w2, b2, w3, b3, wc)


_BP = 2000


def _pool(lin, agg, batch2d, wf, bf):
    grid = N // _BP
    return pl.pallas_call(
        _pool_body,
        grid=(grid,),
        in_specs=[
            pl.BlockSpec((_BP, NOUT), lambda i: (i, 0)),
            pl.BlockSpec((NC, _BP, NOUT), lambda i: (0, i, 0)),
            pl.BlockSpec((_BP, 1), lambda i: (i, 0)),
            _full((NOUT, 10)), _full((1, 10)),
        ],
        out_specs=pl.BlockSpec((NGRAPH, 10), lambda i: (0, 0)),
        out_shape=jax.ShapeDtypeStruct((NGRAPH, 10), jnp.float32),
        scratch_shapes=[pltpu.VMEM((NGRAPH, NOUT), jnp.float32)],
    )(lin, agg, batch2d, wf, bf)


_sc_mesh = plsc.VectorSubcoreMesh(core_axis_name="c", subcore_axis_name="s")


@functools.partial(
    pl.kernel,
    out_type=jax.ShapeDtypeStruct((NC, NP, NOUT), jnp.float32),
    mesh=_sc_mesh,
    compiler_params=pltpu.CompilerParams(use_tc_tiling_on_sc=False),
    scratch_types=[
        pltpu.VMEM((NITER, CH), jnp.int32),
        pltpu.VMEM((NITER, CH), jnp.int32),
        pltpu.VMEM((2 * G, CH, NOUT), jnp.float32),
        pltpu.VMEM((CH, NOUT), jnp.float32),
        pltpu.VMEM_SHARED((NP, NOUT), jnp.float32),
        pltpu.SemaphoreType.DMA,
        pltpu.SemaphoreType.DMA,
    ],
)
def _edge_scatter(y_hbm, src_hbm, dst_hbm, zeros_hbm, out_hbm,
                  sidx, didx, rows, stage, acc, gsem, ssem):
    c = lax.axis_index("c")
    s = lax.axis_index("s")
    wid = s * NC + c

    i1 = pltpu.async_copy(src_hbm.at[wid], sidx, ssem)
    i2 = pltpu.async_copy(dst_hbm.at[wid], didx, ssem)
    pltpu.sync_copy(zeros_hbm, stage)
    zs = [
        pltpu.async_copy(stage, acc.at[pl.ds(s * RPT + k * CH, CH)], gsem)
        for k in range(RPT // CH)
    ]
    i1.wait()
    i2.wait()
    for z in zs:
        z.wait()

    plsc.subcore_barrier()

    R = 2 * G

    def wait_gather():
        pltpu.make_async_copy(y_hbm.at[sidx.at[0]], rows.at[0],
                              gsem).wait()

    def drain_scatter():
        pltpu.make_async_copy(rows.at[0], acc.at[didx.at[0]], ssem).wait()

    for i in range(G):
        pltpu.async_copy(y_hbm.at[sidx.at[i]], rows.at[i], gsem)

    def body(i, _):
        @pl.when(i >= G)
        def _():
            drain_scatter()

        wait_gather()
        pltpu.async_copy(rows.at[lax.rem(i, R)], acc.at[didx.at[i]],
                         ssem, add=True)

        @pl.when(i + G < NITER)
        def _():
            pltpu.async_copy(y_hbm.at[sidx.at[i + G]],
                             rows.at[lax.rem(i + G, R)], gsem)
        return ()

    lax.fori_loop(0, NITER, body, (), unroll=False)
    for _ in range(G):
        drain_scatter()

    plsc.subcore_barrier()

    nk = RPT // CH
    gets = [
        pltpu.async_copy(acc.at[pl.ds(s * RPT + k * CH, CH)],
                         rows.at[k], gsem)
        for k in range(nk)
    ]
    puts = []
    for k in range(nk):
        gets[k].wait()
        puts.append(
            pltpu.async_copy(rows.at[k],
                             out_hbm.at[c, pl.ds(s * RPT + k * CH, CH)],
                             ssem))
    for put in puts:
        put.wait()


def kernel(x, edge_index, batch, Wc1, Wc2, Wc3, W11, b11, W12, b12, W13, b13,
           W21, b21, W22, b22, W23, b23, W31, b31, W32, b32, W33, b33, Wf, bf):
    xp = jnp.pad(x, ((0, NP - N), (0, 0)))
    src = edge_index[0].reshape(NW, NITER, CH)
    dst = edge_index[1].reshape(NW, NITER, CH)
    zeros = jnp.zeros((CH, NOUT), jnp.float32)

    r = lambda b: b.reshape(1, -1)

    lin1, y1 = _layer1(xp, W11, r(b11), W12, r(b12), W13, r(b13), Wc1)
    agg1 = _edge_scatter(y1, src, dst, zeros)
    lin2, y2 = _mid(lin1, agg1, W21, r(b21), W22, r(b22), W23, r(b23), Wc2)
    agg2 = _edge_scatter(y2, src, dst, zeros)
    lin3, y3 = _mid(lin2, agg2, W31, r(b31), W32, r(b32), W33, r(b33), Wc3)
    agg3 = _edge_scatter(y3, src, dst, zeros)
    return _pool(lin3, agg3, batch.reshape(N, 1), Wf, r(bf))


# speedup vs baseline: 1.0017x; 1.0017x over previous
import functools

import jax
import jax.numpy as jnp
from jax import lax
from jax.experimental import pallas as pl
from jax.experimental.pallas import tpu as pltpu
from jax.experimental.pallas import tpu_sc as plsc

N = 10000
E = 320000
D = 128
NOUT = 64
NGRAPH = 128

NC = 2
NS = 16
NW = NC * NS

NP = 10240
RPT = NP // NS
EPW = E // NW
CH = 80
NITER = EPW // CH
G = 5

_HIGH = jax.lax.Precision.DEFAULT


def _dot(a, b):
    return jax.lax.dot(a, b, precision=_HIGH, preferred_element_type=jnp.float32)


def _layer1_body(x_ref, w11, b11, w12, b12, w13, b13, wc, lin_ref, y_ref):
    xb = x_ref[...]
    a = _dot(xb, w11[...]) + b11[...]
    m2 = _dot(xb, w12[...]) + b12[...]
    m3 = _dot(xb, w13[...]) + b13[...]
    lin_ref[...] = a + m2 * m3
    y_ref[...] = _dot(xb, wc[...])


def _mid_body(lin_ref, agg_ref, w1, b1, w2, b2, w3, b3, wc, lin_o, y_o):
    h = jnp.tanh(lin_ref[...] + agg_ref[0] + agg_ref[1])
    a = _dot(h, w1[...]) + b1[...]
    m2 = _dot(h, w2[...]) + b2[...]
    m3 = _dot(h, w3[...]) + b3[...]
    lin_o[...] = a + m2 * m3
    y_o[...] = _dot(h, wc[...])


def _pool_body(lin_ref, agg_ref, batch_ref, wf, bf, out_ref, pooled):
    i = pl.program_id(0)
    h = jnp.tanh(lin_ref[...] + agg_ref[0] + agg_ref[1])
    b = batch_ref[...]
    gids = jax.lax.broadcasted_iota(jnp.int32, (1, NGRAPH), 1)
    onehot = (b == gids).astype(jnp.float32)
    part = jax.lax.dot_general(
        onehot, h, (((0,), (0,)), ((), ())),
        precision=_HIGH, preferred_element_type=jnp.float32)

    @pl.when(i == 0)
    def _():
        pooled[...] = part

    @pl.when(i > 0)
    def _():
        pooled[...] = pooled[...] + part

    @pl.when(i == pl.num_programs(0) - 1)
    def _():
        out_ref[...] = _dot(pooled[...], wf[...]) + bf[...]


_BN = 2048


def _full(shape):
    return pl.BlockSpec(shape, lambda i: (0,) * len(shape))


def _layer1(xp, w11, b11, w12, b12, w13, b13, wc):
    grid = NP // _BN
    return pl.pallas_call(
        _layer1_body,
        grid=(grid,),
        in_specs=[
            pl.BlockSpec((_BN, D), lambda i: (i, 0)),
            _full((D, NOUT)), _full((1, NOUT)),
            _full((D, NOUT)), _full((1, NOUT)),
            _full((D, NOUT)), _full((1, NOUT)),
            _full((D, NOUT)),
        ],
        out_specs=[
            pl.BlockSpec((_BN, NOUT), lambda i: (i, 0)),
            pl.BlockSpec((_BN, NOUT), lambda i: (i, 0)),
        ],
        out_shape=[
            jax.ShapeDtypeStruct((NP, NOUT), jnp.float32),
            jax.ShapeDtypeStruct((NP, NOUT), jnp.float32),
        ],
    )(xp, w11, b11, w12, b12, w13, b13, wc)


def _mid(lin, agg, w1, b1, w2, b2, w3, b3, wc):
    grid = NP // _BN
    return pl.pallas_call(
        _mid_body,
        grid=(grid,),
        in_specs=[
            pl.BlockSpec((_BN, NOUT), lambda i: (i, 0)),
            pl.BlockSpec((NC, _BN, NOUT), lambda i: (0, i, 0)),
            _full((NOUT, NOUT)), _full((1, NOUT)),
            _full((NOUT, NOUT)), _full((1, NOUT)),
            _full((NOUT, NOUT)), _full((1, NOUT)),
            _full((NOUT, NOUT)),
        ],
        out_specs=[
            pl.BlockSpec((_BN, NOUT), lambda i: (i, 0)),
            pl.BlockSpec((_BN, NOUT), lambda i: (i, 0)),
        ],
        out_shape=[
            jax.ShapeDtypeStruct((NP, NOUT), jnp.float32),
            jax.ShapeDtypeStruct((NP, NOUT), jnp.float32),
        ],
    )(lin, agg, w1, b1, w2, b2, w3, b3, wc)


_BP = 2000


def _pool(lin, agg, batch2d, wf, bf):
    grid = N // _BP
    return pl.pallas_call(
        _pool_body,
        grid=(grid,),
        in_specs=[
            pl.BlockSpec((_BP, NOUT), lambda i: (i, 0)),
            pl.BlockSpec((NC, _BP, NOUT), lambda i: (0, i, 0)),
            pl.BlockSpec((_BP, 1), lambda i: (i, 0)),
            _full((NOUT, 10)), _full((1, 10)),
        ],
        out_specs=pl.BlockSpec((NGRAPH, 10), lambda i: (0, 0)),
        out_shape=jax.ShapeDtypeStruct((NGRAPH, 10), jnp.float32),
        scratch_shapes=[pltpu.VMEM((NGRAPH, NOUT), jnp.float32)],
    )(lin, agg, batch2d, wf, bf)


_sc_mesh = plsc.VectorSubcoreMesh(core_axis_name="c", subcore_axis_name="s")


@functools.partial(
    pl.kernel,
    out_type=jax.ShapeDtypeStruct((NC, NP, NOUT), jnp.float32),
    mesh=_sc_mesh,
    compiler_params=pltpu.CompilerParams(use_tc_tiling_on_sc=False),
    scratch_types=[
        pltpu.VMEM((NITER, CH), jnp.int32),
        pltpu.VMEM((NITER, CH), jnp.int32),
        pltpu.VMEM((2 * G, CH, NOUT), jnp.float32),
        pltpu.VMEM((CH, NOUT), jnp.float32),
        pltpu.VMEM_SHARED((NP, NOUT), jnp.float32),
        pltpu.SemaphoreType.DMA,
        pltpu.SemaphoreType.DMA,
    ],
)
def _edge_scatter(y_hbm, src_hbm, dst_hbm, zeros_hbm, out_hbm,
                  sidx, didx, rows, stage, acc, gsem, ssem):
    c = lax.axis_index("c")
    s = lax.axis_index("s")
    wid = s * NC + c

    i1 = pltpu.async_copy(src_hbm.at[wid], sidx, ssem)
    i2 = pltpu.async_copy(dst_hbm.at[wid], didx, ssem)
    pltpu.sync_copy(zeros_hbm, stage)
    zs = [
        pltpu.async_copy(stage, acc.at[pl.ds(s * RPT + k * CH, CH)], gsem)
        for k in range(RPT // CH)
    ]
    i1.wait()
    i2.wait()
    for z in zs:
        z.wait()

    plsc.subcore_barrier()

    R = 2 * G

    def wait_gather():
        pltpu.make_async_copy(y_hbm.at[sidx.at[0]], rows.at[0],
                              gsem).wait()

    def drain_scatter():
        pltpu.make_async_copy(rows.at[0], acc.at[didx.at[0]], ssem).wait()

    for i in range(G):
        pltpu.async_copy(y_hbm.at[sidx.at[i]], rows.at[i], gsem)

    def body(i, _):
        @pl.when(i >= G)
        def _():
            drain_scatter()

        wait_gather()
        pltpu.async_copy(rows.at[lax.rem(i, R)], acc.at[didx.at[i]],
                         ssem, add=True)

        @pl.when(i + G < NITER)
        def _():
            pltpu.async_copy(y_hbm.at[sidx.at[i + G]],
                             rows.at[lax.rem(i + G, R)], gsem)
        return ()

    lax.fori_loop(0, NITER, body, (), unroll=False)
    for _ in range(G):
        drain_scatter()

    plsc.subcore_barrier()

    nk = RPT // CH
    gets = [
        pltpu.async_copy(acc.at[pl.ds(s * RPT + k * CH, CH)],
                         rows.at[k], gsem)
        for k in range(nk)
    ]
    puts = []
    for k in range(nk):
        gets[k].wait()
        puts.append(
            pltpu.async_copy(rows.at[k],
                             out_hbm.at[c, pl.ds(s * RPT + k * CH, CH)],
                             ssem))
    for put in puts:
        put.wait()


def kernel(x, edge_index, batch, Wc1, Wc2, Wc3, W11, b11, W12, b12, W13, b13,
           W21, b21, W22, b22, W23, b23, W31, b31, W32, b32, W33, b33, Wf, bf):
    xp = jnp.pad(x, ((0, NP - N), (0, 0)))
    src = edge_index[0].reshape(NW, NITER, CH)
    dst = edge_index[1].reshape(NW, NITER, CH)
    zeros = jnp.zeros((CH, NOUT), jnp.float32)

    r = lambda b: b.reshape(1, -1)

    lin1, y1 = _layer1(xp, W11, r(b11), W12, r(b12), W13, r(b13), Wc1)
    agg1 = _edge_scatter(y1, src, dst, zeros)
    lin2, y2 = _mid(lin1, agg1, W21, r(b21), W22, r(b22), W23, r(b23), Wc2)
    agg2 = _edge_scatter(y2, src, dst, zeros)
    lin3, y3 = _mid(lin2, agg2, W31, r(b31), W32, r(b32), W33, r(b33), Wc3)
    agg3 = _edge_scatter(y3, src, dst, zeros)
    return _pool(lin3, agg3, batch.reshape(N, 1), Wf, r(bf))
